# Initial kernel scaffold; baseline (speedup 1.0000x reference)
import jax
import jax.numpy as jnp
from jax.experimental import pallas as pl


def kernel(feature, _target, anchor):
    raise NotImplementedError("write your pallas kernel here")


# speedup vs baseline: 4.5926x; 4.5926x over previous
import functools

import jax
import jax.numpy as jnp
from jax import lax
from jax.experimental import pallas as pl
from jax.experimental.pallas import tpu as pltpu
from jax.experimental.pallas import tpu_sc as plsc

CLS = 100
CLS_PAD = 128
FEAT = 128
BATCH = 16384
LAM = 0.1

NC = 2
NS = 16
NW = NC * NS
B_PER_W = BATCH // NW

_mesh = plsc.VectorSubcoreMesh(core_axis_name="c", subcore_axis_name="s")


@functools.partial(
    pl.kernel,
    out_type=jax.ShapeDtypeStruct((NW, CLS_PAD), jnp.float32),
    mesh=_mesh,
    compiler_params=pltpu.CompilerParams(needs_layout_passes=False),
    scratch_types=[
        pltpu.VMEM((B_PER_W + 16,), jnp.int32),
        pltpu.VMEM((CLS_PAD,), jnp.float32),
    ],
)
def _count_kernel(idx_hbm, out_hbm, idx_v, cnt_v):
    wid = lax.axis_index("s") * NC + lax.axis_index("c")
    base = wid * B_PER_W
    pltpu.sync_copy(idx_hbm.at[pl.ds(base, B_PER_W)], idx_v.at[pl.ds(0, B_PER_W)])
    for c in range(CLS_PAD // 16):
        cnt_v[pl.ds(16 * c, 16)] = jnp.zeros((16,), jnp.float32)
    lane0 = lax.iota(jnp.int32, 16) == 0
    ones = jnp.ones((16,), jnp.float32)

    def body(s, carry):
        t = idx_v[pl.ds(s, 16)][0]
        tvec = jnp.full((16,), t, jnp.int32)
        plsc.addupdate_scatter(cnt_v, [tvec], ones, mask=lane0)
        return carry

    lax.fori_loop(0, B_PER_W, body, 0)
    pltpu.sync_copy(cnt_v, out_hbm.at[wid])


@functools.partial(
    pl.kernel,
    out_type=jax.ShapeDtypeStruct((NW, 16), jnp.float32),
    mesh=_mesh,
    scratch_types=[
        pltpu.VMEM((B_PER_W * FEAT,), jnp.float32),
        pltpu.VMEM((B_PER_W + 16,), jnp.int32),
        pltpu.VMEM((CLS_PAD * FEAT,), jnp.float32),
        pltpu.VMEM((CLS_PAD,), jnp.float32),
        pltpu.VMEM((16,), jnp.float32),
    ],
)
def _main_kernel(feat_hbm, idx_hbm, anc_hbm, wts_hbm, out_hbm,
                 feat_v, idx_v, anc_v, wts_v, out_v):
    wid = lax.axis_index("s") * NC + lax.axis_index("c")
    base = wid * B_PER_W
    pltpu.sync_copy(feat_hbm.at[pl.ds(base * FEAT, B_PER_W * FEAT)], feat_v)
    pltpu.sync_copy(idx_hbm.at[pl.ds(base, B_PER_W)], idx_v.at[pl.ds(0, B_PER_W)])
    pltpu.sync_copy(anc_hbm, anc_v)
    pltpu.sync_copy(wts_hbm, wts_v)

    def body(s, grand):
        t = idx_v[pl.ds(s, 16)][0]
        w = wts_v[pl.ds(t, 16)][0]
        frow = s * FEAT
        arow = t * FEAT
        acc = None
        for c in range(FEAT // 16):
            f = feat_v[pl.ds(frow + 16 * c, 16)]
            a = anc_v[pl.ds(arow + 16 * c, 16)]
            d = f - a
            p = d * d
            acc = p if acc is None else acc + p
        return grand + w * acc

    grand = lax.fori_loop(0, B_PER_W, body, jnp.zeros((16,), jnp.float32))
    out_v[...] = grand
    pltpu.sync_copy(out_v, out_hbm.at[wid])


def kernel(feature, _target, anchor):
    idx = _target.astype(jnp.int32)
    partial_counts = _count_kernel(idx)
    counts = jnp.sum(partial_counts, axis=0)
    wts = jnp.where(counts > 0, 1.0 / counts, 0.0)
    anc = jnp.pad(anchor, ((0, CLS_PAD - CLS), (0, 0))).reshape(-1)
    partials = _main_kernel(feature.reshape(-1), idx, anc, wts)
    return (LAM / CLS) * jnp.sum(partials)


# speedup vs baseline: 4.7584x; 1.0361x over previous
import functools

import jax
import jax.numpy as jnp
from jax import lax
from jax.experimental import pallas as pl
from jax.experimental.pallas import tpu as pltpu
from jax.experimental.pallas import tpu_sc as plsc

CLS = 100
CLS_PAD = 128
FEAT = 128
BATCH = 16384
LAM = 0.1

NC = 2
NS = 16
NW = NC * NS
B_PER_W = BATCH // NW
B_BIN = BATCH // NS

_mesh = plsc.VectorSubcoreMesh(core_axis_name="c", subcore_axis_name="s")


@functools.partial(
    pl.kernel,
    out_type=jax.ShapeDtypeStruct((NW, 16), jnp.float32),
    mesh=_mesh,
    compiler_params=pltpu.CompilerParams(needs_layout_passes=False),
    scratch_types=[
        pltpu.VMEM((B_PER_W * FEAT,), jnp.float32),
        pltpu.VMEM((B_PER_W + 16,), jnp.int32),
        pltpu.VMEM((B_BIN + 16,), jnp.int32),
        pltpu.VMEM((CLS_PAD * FEAT,), jnp.float32),
        pltpu.VMEM((CLS_PAD,), jnp.float32),
        pltpu.VMEM((NS, CLS_PAD), jnp.float32),
        pltpu.VMEM_SHARED((NS, CLS_PAD), jnp.float32),
        pltpu.VMEM((16,), jnp.float32),
        pltpu.SemaphoreType.DMA,
    ],
)
def _anchor_loss_kernel(feat_hbm, idx_hbm, anc_hbm, out_hbm,
                        feat_v, idx_v, bin_v, anc_v, wts_v, sums_v,
                        shared_cnt, out_v, sem):
    sid = lax.axis_index("s")
    wid = sid * NC + lax.axis_index("c")
    base = wid * B_PER_W

    fcopy = pltpu.async_copy(
        feat_hbm.at[pl.ds(base * FEAT, B_PER_W * FEAT)], feat_v, sem)

    pltpu.sync_copy(idx_hbm.at[pl.ds(sid * B_BIN, B_BIN)],
                    bin_v.at[pl.ds(0, B_BIN)])
    pltpu.sync_copy(idx_hbm.at[pl.ds(base, B_PER_W)],
                    idx_v.at[pl.ds(0, B_PER_W)])
    pltpu.sync_copy(anc_hbm, anc_v)
    for c in range(CLS_PAD // 16):
        wts_v[pl.ds(16 * c, 16)] = jnp.zeros((16,), jnp.float32)
    lane0 = lax.iota(jnp.int32, 16) == 0
    ones = jnp.ones((16,), jnp.float32)

    def bin_body(s, carry):
        t = bin_v[pl.ds(s, 16)][0]
        tvec = jnp.full((16,), t, jnp.int32)
        plsc.addupdate_scatter(wts_v, [tvec], ones, mask=lane0)
        return carry

    lax.fori_loop(0, B_BIN, bin_body, 0)

    pltpu.sync_copy(wts_v, shared_cnt.at[sid])
    plsc.subcore_barrier()
    pltpu.sync_copy(shared_cnt, sums_v)
    for c in range(CLS_PAD // 16):
        tot = sums_v[0, pl.ds(16 * c, 16)]
        for r in range(1, NS):
            tot = tot + sums_v[r, pl.ds(16 * c, 16)]
        w = jnp.where(tot > 0.0, 1.0 / tot, 0.0)
        wts_v[pl.ds(16 * c, 16)] = w

    fcopy.wait()

    def body(s, grand):
        t = idx_v[pl.ds(s, 16)][0]
        w = wts_v[pl.ds(t, 16)][0]
        frow = s * FEAT
        arow = t * FEAT
        acc = None
        for c in range(FEAT // 16):
            f = feat_v[pl.ds(frow + 16 * c, 16)]
            a = anc_v[pl.ds(arow + 16 * c, 16)]
            d = f - a
            p = d * d
            acc = p if acc is None else acc + p
        return grand + w * acc

    grand = lax.fori_loop(0, B_PER_W, body, jnp.zeros((16,), jnp.float32))
    out_v[...] = grand
    pltpu.sync_copy(out_v, out_hbm.at[wid])


def kernel(feature, _target, anchor):
    idx = _target.astype(jnp.int32)
    anc = jnp.pad(anchor, ((0, CLS_PAD - CLS), (0, 0))).reshape(-1)
    partials = _anchor_loss_kernel(feature.reshape(-1), idx, anc)
    return (LAM / CLS) * jnp.sum(partials)
